# baseline (speedup 1.0000x reference)
import functools

import jax
import jax.numpy as jnp
from jax import lax
from jax.experimental import pallas as pl
from jax.experimental.pallas import tpu as pltpu
from jax.experimental.pallas import tpu_sc as plsc

N_NODES = 10000
N_EDGES = 320000
D_FEAT = 128

NC = 2
NS = 16
DH = D_FEAT // NC

EPT = N_EDGES // NS
K = 100
NCHUNK = EPT // K
WRB = 640
WRB_LAST = N_NODES - (NS - 1) * WRB
DEG_W = 8

_mesh = plsc.VectorSubcoreMesh(core_axis_name="c", subcore_axis_name="s")


@functools.partial(
    pl.kernel,
    out_type=(
        jax.ShapeDtypeStruct((NC, N_NODES, DH), jnp.float32),
        jax.ShapeDtypeStruct((N_NODES, DEG_W), jnp.float32),
    ),
    mesh=_mesh,
    compiler_params=pltpu.CompilerParams(use_tc_tiling_on_sc=False),
    scratch_types=[
        pltpu.VMEM((NCHUNK, K), jnp.int32),
        pltpu.VMEM((NCHUNK, K), jnp.int32),
        pltpu.VMEM((K, DH), jnp.float32),
        pltpu.VMEM((K, DH), jnp.float32),
        pltpu.VMEM((K, DEG_W), jnp.float32),
        pltpu.VMEM_SHARED((N_NODES, DH), jnp.float32),
        pltpu.VMEM_SHARED((N_NODES, DEG_W), jnp.float32),
        pltpu.SemaphoreType.DMA,
        pltpu.SemaphoreType.DMA,
        pltpu.SemaphoreType.DMA,
        pltpu.SemaphoreType.DMA,
        pltpu.SemaphoreType.DMA,
        pltpu.SemaphoreType.DMA,
    ],
)
def _sc_agg(x0_hbm, x1_hbm, src_hbm, dst_hbm, zrow_hbm, zdeg_hbm, ones_hbm,
            out_hbm, deg_out_hbm,
            src_v, dst_v, rows0_v, rows1_v, ones_v, acc_sh, deg_sh,
            sem_g0, sem_g1, sem_s0, sem_s1, sem_d0, sem_d1):
    c = lax.axis_index("c")
    s = lax.axis_index("s")

    @pl.when(s < NS - 1)
    def _():
        pltpu.sync_copy(zrow_hbm, acc_sh.at[pl.ds(s * WRB, WRB)])
        pltpu.sync_copy(zdeg_hbm, deg_sh.at[pl.ds(s * WRB, WRB)])

    @pl.when(s == NS - 1)
    def _():
        pltpu.sync_copy(zrow_hbm.at[pl.ds(0, WRB_LAST)],
                        acc_sh.at[pl.ds((NS - 1) * WRB, WRB_LAST)])
        pltpu.sync_copy(zdeg_hbm.at[pl.ds(0, WRB_LAST)],
                        deg_sh.at[pl.ds((NS - 1) * WRB, WRB_LAST)])

    pltpu.sync_copy(ones_hbm, ones_v)
    pltpu.sync_copy(src_hbm.at[s], src_v)
    pltpu.sync_copy(dst_hbm.at[s], dst_v)
    plsc.subcore_barrier()

    def issue_gather(j, buf, sem_):
        @pl.when(c == 0)
        def _():
            pltpu.async_copy(x0_hbm.at[src_v.at[j]], buf, sem_)

        @pl.when(c == 1)
        def _():
            pltpu.async_copy(x1_hbm.at[src_v.at[j]], buf, sem_)

    def step(j, buf_a, sem_ga, sem_sa, sem_da, buf_b, sem_gb, sem_sb, sem_db):
        pltpu.make_async_copy(x0_hbm.at[src_v.at[j]], buf_a, sem_ga).wait()

        @pl.when(j >= 1)
        def _():
            pltpu.make_async_copy(buf_b, acc_sh.at[dst_v.at[j]],
                                  sem_sb).wait()

            @pl.when(c == 0)
            def _():
                pltpu.make_async_copy(ones_v, deg_sh.at[dst_v.at[j]],
                                      sem_db).wait()

        @pl.when(j + 1 < NCHUNK)
        def _():
            issue_gather(j + 1, buf_b, sem_gb)

        pltpu.async_copy(buf_a, acc_sh.at[dst_v.at[j]], sem_sa, add=True)

        @pl.when(c == 0)
        def _():
            pltpu.async_copy(ones_v, deg_sh.at[dst_v.at[j]], sem_da, add=True)

    issue_gather(0, rows0_v, sem_g0)

    def chunk(j, carry):
        @pl.when(j % 2 == 0)
        def _():
            step(j, rows0_v, sem_g0, sem_s0, sem_d0,
                 rows1_v, sem_g1, sem_s1, sem_d1)

        @pl.when(j % 2 == 1)
        def _():
            step(j, rows1_v, sem_g1, sem_s1, sem_d1,
                 rows0_v, sem_g0, sem_s0, sem_d0)

        return carry

    lax.fori_loop(0, NCHUNK, chunk, 0)

    pltpu.make_async_copy(rows1_v, acc_sh.at[dst_v.at[NCHUNK - 1]],
                          sem_s1).wait()

    @pl.when(c == 0)
    def _():
        pltpu.make_async_copy(ones_v, deg_sh.at[dst_v.at[NCHUNK - 1]],
                              sem_d1).wait()

    plsc.subcore_barrier()

    @pl.when(s < NS - 1)
    def _():
        pltpu.sync_copy(acc_sh.at[pl.ds(s * WRB, WRB)],
                        out_hbm.at[c, pl.ds(s * WRB, WRB)])

        @pl.when(c == 0)
        def _():
            pltpu.sync_copy(deg_sh.at[pl.ds(s * WRB, WRB)],
                            deg_out_hbm.at[pl.ds(s * WRB, WRB)])

    @pl.when(s == NS - 1)
    def _():
        pltpu.sync_copy(acc_sh.at[pl.ds((NS - 1) * WRB, WRB_LAST)],
                        out_hbm.at[c, pl.ds((NS - 1) * WRB, WRB_LAST)])

        @pl.when(c == 0)
        def _():
            pltpu.sync_copy(deg_sh.at[pl.ds((NS - 1) * WRB, WRB_LAST)],
                            deg_out_hbm.at[pl.ds((NS - 1) * WRB, WRB_LAST)])


_ROWS_BLK = 1000


def _combine_body(p_ref, d_ref, o_ref):
    inv = 1.0 / jnp.clip(d_ref[:, 0], 1.0, None)[:, None]
    o_ref[:, 0:DH] = p_ref[0] * inv
    o_ref[:, DH:D_FEAT] = p_ref[1] * inv


def _combine(partial, deg8):
    return pl.pallas_call(
        _combine_body,
        out_shape=jax.ShapeDtypeStruct((N_NODES, D_FEAT), jnp.float32),
        grid=(N_NODES // _ROWS_BLK,),
        in_specs=[
            pl.BlockSpec((NC, _ROWS_BLK, DH), lambda i: (0, i, 0)),
            pl.BlockSpec((_ROWS_BLK, DEG_W), lambda i: (i, 0)),
        ],
        out_specs=pl.BlockSpec((_ROWS_BLK, D_FEAT), lambda i: (i, 0)),
    )(partial, deg8)


def kernel(x, edge_index):
    x0 = x[:, :DH]
    x1 = x[:, DH:]
    src3 = edge_index[0].reshape(NS, NCHUNK, K)
    dst3 = edge_index[1].reshape(NS, NCHUNK, K)
    zrow = jnp.zeros((WRB, DH), jnp.float32)
    zdeg = jnp.zeros((WRB, DEG_W), jnp.float32)
    ones = jnp.ones((K, DEG_W), jnp.float32)
    partial, deg8 = _sc_agg(x0, x1, src3, dst3, zrow, zdeg, ones)
    return _combine(partial, deg8)


# speedup vs baseline: 1.0548x; 1.0548x over previous
import functools

import jax
import jax.numpy as jnp
from jax import lax
from jax.experimental import pallas as pl
from jax.experimental.pallas import tpu as pltpu
from jax.experimental.pallas import tpu_sc as plsc

N_NODES = 10000
N_EDGES = 320000
D_FEAT = 128

NC = 2
NS = 16
DH = D_FEAT // NC

EPT = N_EDGES // NS
K = 80
NCHUNK = EPT // K
WRB = 640
WRB_LAST = N_NODES - (NS - 1) * WRB
DEG_W = 8

_mesh = plsc.VectorSubcoreMesh(core_axis_name="c", subcore_axis_name="s")


@functools.partial(
    pl.kernel,
    out_type=(
        jax.ShapeDtypeStruct((NC, N_NODES, DH), jnp.float32),
        jax.ShapeDtypeStruct((N_NODES, DEG_W), jnp.float32),
    ),
    mesh=_mesh,
    compiler_params=pltpu.CompilerParams(use_tc_tiling_on_sc=False),
    scratch_types=[
        pltpu.VMEM((NCHUNK, K), jnp.int32),
        pltpu.VMEM((NCHUNK, K), jnp.int32),
        pltpu.VMEM((NCHUNK, K), jnp.int32),
        pltpu.VMEM((K, DH), jnp.float32),
        pltpu.VMEM((K, DH), jnp.float32),
        pltpu.VMEM((K, DEG_W), jnp.float32),
        pltpu.VMEM_SHARED((N_NODES, DH), jnp.float32),
        pltpu.VMEM_SHARED((N_NODES, DEG_W), jnp.float32),
        pltpu.SemaphoreType.DMA,
        pltpu.SemaphoreType.DMA,
        pltpu.SemaphoreType.DMA,
        pltpu.SemaphoreType.DMA,
        pltpu.SemaphoreType.DMA,
        pltpu.SemaphoreType.DMA,
    ],
)
def _sc_agg(x2_hbm, e4_hbm, zrow_hbm, zdeg_hbm, ones_hbm,
            out_hbm, deg_out_hbm,
            src_v, src2_v, dst_v, rows0_v, rows1_v, ones_v, acc_sh, deg_sh,
            sem_g0, sem_g1, sem_s0, sem_s1, sem_d0, sem_d1):
    c = lax.axis_index("c")
    s = lax.axis_index("s")

    @pl.when(s < NS - 1)
    def _():
        pltpu.sync_copy(zrow_hbm, acc_sh.at[pl.ds(s * WRB, WRB)])
        pltpu.sync_copy(zdeg_hbm, deg_sh.at[pl.ds(s * WRB, WRB)])

    @pl.when(s == NS - 1)
    def _():
        pltpu.sync_copy(zrow_hbm.at[pl.ds(0, WRB_LAST)],
                        acc_sh.at[pl.ds((NS - 1) * WRB, WRB_LAST)])
        pltpu.sync_copy(zdeg_hbm.at[pl.ds(0, WRB_LAST)],
                        deg_sh.at[pl.ds((NS - 1) * WRB, WRB_LAST)])

    pltpu.sync_copy(ones_hbm, ones_v)
    pltpu.sync_copy(e4_hbm.at[0, s], src_v)
    pltpu.sync_copy(e4_hbm.at[1, s], dst_v)
    plsc.subcore_barrier()

    cc = c * 1

    def xform_row(j):
        for i in range(K // 16):
            v = src_v[j, pl.ds(i * 16, 16)]
            src2_v[j, pl.ds(i * 16, 16)] = v * 2 + cc

    def issue_gather(j, buf, sem_):
        pltpu.async_copy(x2_hbm.at[src2_v.at[j]], buf, sem_)

    def step(j, buf_a, sem_ga, sem_sa, sem_da, buf_b, sem_gb, sem_sb, sem_db):
        pltpu.make_async_copy(x2_hbm.at[src2_v.at[j]], buf_a, sem_ga).wait()

        @pl.when(j >= 1)
        def _():
            pltpu.make_async_copy(buf_b, acc_sh.at[dst_v.at[j]],
                                  sem_sb).wait()
            pltpu.make_async_copy(ones_v, deg_sh.at[dst_v.at[j]],
                                  sem_db).wait()

        @pl.when(j + 1 < NCHUNK)
        def _():
            xform_row(j + 1)
            issue_gather(j + 1, buf_b, sem_gb)

        pltpu.async_copy(buf_a, acc_sh.at[dst_v.at[j]], sem_sa, add=True)
        pltpu.async_copy(ones_v, deg_sh.at[dst_v.at[j]], sem_da, add=True)

    xform_row(0)
    issue_gather(0, rows0_v, sem_g0)

    def chunk(j, carry):
        @pl.when(j % 2 == 0)
        def _():
            step(j, rows0_v, sem_g0, sem_s0, sem_d0,
                 rows1_v, sem_g1, sem_s1, sem_d1)

        @pl.when(j % 2 == 1)
        def _():
            step(j, rows1_v, sem_g1, sem_s1, sem_d1,
                 rows0_v, sem_g0, sem_s0, sem_d0)

        return carry

    lax.fori_loop(0, NCHUNK, chunk, 0)

    pltpu.make_async_copy(rows1_v, acc_sh.at[dst_v.at[NCHUNK - 1]],
                          sem_s1).wait()
    pltpu.make_async_copy(ones_v, deg_sh.at[dst_v.at[NCHUNK - 1]],
                          sem_d1).wait()

    plsc.subcore_barrier()

    @pl.when(s < NS - 1)
    def _():
        pltpu.sync_copy(acc_sh.at[pl.ds(s * WRB, WRB)],
                        out_hbm.at[c, pl.ds(s * WRB, WRB)])

        @pl.when(c == 0)
        def _():
            pltpu.sync_copy(deg_sh.at[pl.ds(s * WRB, WRB)],
                            deg_out_hbm.at[pl.ds(s * WRB, WRB)])

    @pl.when(s == NS - 1)
    def _():
        pltpu.sync_copy(acc_sh.at[pl.ds((NS - 1) * WRB, WRB_LAST)],
                        out_hbm.at[c, pl.ds((NS - 1) * WRB, WRB_LAST)])

        @pl.when(c == 0)
        def _():
            pltpu.sync_copy(deg_sh.at[pl.ds((NS - 1) * WRB, WRB_LAST)],
                            deg_out_hbm.at[pl.ds((NS - 1) * WRB, WRB_LAST)])


_ROWS_BLK = 1000


def _combine_body(p_ref, d_ref, o_ref):
    inv = 1.0 / jnp.clip(d_ref[:, 0], 1.0, None)[:, None]
    o_ref[:, 0:DH] = p_ref[0] * inv
    o_ref[:, DH:D_FEAT] = p_ref[1] * inv


def _combine(partial, deg8):
    return pl.pallas_call(
        _combine_body,
        out_shape=jax.ShapeDtypeStruct((N_NODES, D_FEAT), jnp.float32),
        grid=(N_NODES // _ROWS_BLK,),
        in_specs=[
            pl.BlockSpec((NC, _ROWS_BLK, DH), lambda i: (0, i, 0)),
            pl.BlockSpec((_ROWS_BLK, DEG_W), lambda i: (i, 0)),
        ],
        out_specs=pl.BlockSpec((_ROWS_BLK, D_FEAT), lambda i: (i, 0)),
    )(partial, deg8)


def kernel(x, edge_index):
    x2 = x.reshape(2 * N_NODES, DH)
    e4 = edge_index.reshape(2, NS, NCHUNK, K)
    zrow = jnp.zeros((WRB, DH), jnp.float32)
    zdeg = jnp.zeros((WRB, DEG_W), jnp.float32)
    ones = jnp.ones((K, DEG_W), jnp.float32)
    partial, deg8 = _sc_agg(x2, e4, zrow, zdeg, ones)
    return _combine(partial, deg8)
